# Initial kernel scaffold; baseline (speedup 1.0000x reference)
import jax
import jax.numpy as jnp
from jax.experimental import pallas as pl


def kernel(x, edge_index, edge_weight, node_type, W0, tb0, W1, tb1):
    raise NotImplementedError("write your pallas kernel here")


# speedup vs baseline: 2.5469x; 2.5469x over previous
import functools

import jax
import jax.numpy as jnp
from jax import lax
from jax.experimental import pallas as pl
from jax.experimental.pallas import tpu as pltpu
from jax.experimental.pallas import tpu_sc as plsc

N = 10000
E = 320000
D = 128
T = 8

NC = 2
NS = 16
NW = NC * NS
LANES = 16
G = 128
NG = -(-E // (NW * G))
NG += NG % 2
EPT = NG * G
NP = 10240
ROWS_PER_TILE = NP // NS
DRAIN_CHUNK = 128
N_DRAIN = ROWS_PER_TILE // DRAIN_CHUNK

MM_BLOCK = 1000
MM_GRID = N // MM_BLOCK
TC_BLOCK = 1024
TC_GRID = NP // TC_BLOCK


_GATHER_DNUMS = lax.GatherDimensionNumbers(
    offset_dims=(), collapsed_slice_dims=(0,), start_index_map=(0,))


def _lane_bcast(v, lane):
    idx = jnp.full((LANES, 1), lane, jnp.int32)
    return lax.gather(v, idx, _GATHER_DNUMS, slice_sizes=(1,),
                      mode=lax.GatherScatterMode.PROMISE_IN_BOUNDS)


def _sc_scatter_spmem_body(h_hbm, src_hbm, dst_hbm, w_hbm, out_hbm,
                           src_v, dst_v, w_v, buf_a, buf_b, acc,
                           sem_a, sem_b, sem_i):
    cid = lax.axis_index("c")
    sid = lax.axis_index("s")
    wid = sid * NC + cid

    def _idx_fetch(g, slot, sem=None):
        for hbm, v in ((src_hbm, src_v), (dst_hbm, dst_v), (w_hbm, w_v)):
            if sem is None:
                pltpu.sync_copy(hbm.at[wid, g], v.at[slot])
            else:
                pltpu.async_copy(hbm.at[wid, g], v.at[slot], sem)

    def _idx_drain(g, slot):
        for hbm, v in ((src_hbm, src_v), (dst_hbm, dst_v), (w_hbm, w_v)):
            pltpu.make_async_copy(hbm.at[wid, g], v.at[slot], sem_i).wait()

    _idx_fetch(0, 0)
    pltpu.async_copy(h_hbm.at[src_v.at[0]], buf_a, sem_a)
    _idx_fetch(1, 1, sem_i)

    def _zero_row(r, _):
        for j in range(D // LANES):
            buf_b[r, pl.ds(j * LANES, LANES)] = jnp.zeros((LANES,), jnp.float32)
        return _
    lax.fori_loop(0, DRAIN_CHUNK, _zero_row, None)
    for k in range(N_DRAIN):
        pltpu.sync_copy(buf_b.at[pl.ds(0, DRAIN_CHUNK)],
                        acc.at[pl.ds(sid * ROWS_PER_TILE + k * DRAIN_CHUNK,
                                     DRAIN_CHUNK)])
    plsc.subcore_barrier()

    def _scale(buf, slot):
        def _sub(q, _):
            wv = w_v[slot, pl.ds(q * LANES, LANES)]
            for l in range(LANES):
                wb = _lane_bcast(wv, l)
                e = q * LANES + l
                for j in range(D // LANES):
                    sl = pl.ds(j * LANES, LANES)
                    buf[e, sl] = buf[e, sl] * wb
            return _
        lax.fori_loop(0, G // LANES, _sub, None)

    def _half(g, slot, buf, sem, obuf, osem):
        _idx_drain(g + 1, 1 - slot)
        pltpu.async_copy(h_hbm.at[src_v.at[1 - slot]], obuf, osem)
        pltpu.make_async_copy(h_hbm.at[src_v.at[slot]], buf, sem).wait()
        _scale(buf, slot)
        pltpu.sync_copy(buf, acc.at[dst_v.at[slot]], add=True)
        _idx_fetch(g + 2, slot, sem_i)

    def _step(g, _):
        _half(g, 0, buf_a, sem_a, buf_b, sem_b)
        _half(g + 1, 1, buf_b, sem_b, buf_a, sem_a)
        return _
    lax.fori_loop(0, NG // 2, lambda i, c: _step(i * 2, c), None)

    pltpu.make_async_copy(h_hbm.at[src_v.at[0]], buf_a, sem_a).wait()
    _idx_drain(NG + 1, 1)
    plsc.subcore_barrier()

    for k in range(N_DRAIN):
        rows = pl.ds(sid * ROWS_PER_TILE + k * DRAIN_CHUNK, DRAIN_CHUNK)
        pltpu.sync_copy(acc.at[rows], buf_a.at[pl.ds(0, DRAIN_CHUNK)])
        pltpu.sync_copy(buf_a.at[pl.ds(0, DRAIN_CHUNK)], out_hbm.at[cid, rows])


_sc_mesh = plsc.VectorSubcoreMesh(core_axis_name="c", subcore_axis_name="s",
                                  num_cores=NC, num_subcores=NS)

_sc_scatter = pl.kernel(
    _sc_scatter_spmem_body,
    out_type=jax.ShapeDtypeStruct((NC, NP, D), jnp.float32),
    mesh=_sc_mesh,
    scratch_types=[
        pltpu.VMEM((2, G), jnp.int32),
        pltpu.VMEM((2, G), jnp.int32),
        pltpu.VMEM((2, G), jnp.float32),
        pltpu.VMEM((G, D), jnp.float32),
        pltpu.VMEM((G, D), jnp.float32),
        pltpu.VMEM_SHARED((NP, D), jnp.float32),
        pltpu.SemaphoreType.DMA,
        pltpu.SemaphoreType.DMA,
        pltpu.SemaphoreType.DMA,
    ],
)


def _mm_body(x_ref, w_ref, o_ref):
    o_ref[...] = jnp.dot(x_ref[...], w_ref[...],
                         preferred_element_type=jnp.float32)


_mm = pl.pallas_call(
    _mm_body,
    grid=(MM_GRID,),
    in_specs=[pl.BlockSpec((MM_BLOCK, D), lambda i: (i, 0)),
              pl.BlockSpec((D, D), lambda i: (0, 0))],
    out_specs=pl.BlockSpec((MM_BLOCK, D), lambda i: (i, 0)),
    out_shape=jax.ShapeDtypeStruct((N, D), jnp.float32),
)


def _bias_block(nt_ref, tb_ref):
    nt = nt_ref[0, 0, :]
    oh = (nt[:, None] == lax.broadcasted_iota(jnp.int32, (TC_BLOCK, T), 1))
    return jnp.dot(oh.astype(jnp.float32), tb_ref[...],
                   preferred_element_type=jnp.float32)


def _fuse_body(p_ref, nt_ref, tb_ref, w_ref, o_ref):
    h = p_ref[0] + p_ref[1] + _bias_block(nt_ref, tb_ref)
    h = jnp.maximum(h, 0.0)
    o_ref[...] = jnp.dot(h, w_ref[...], preferred_element_type=jnp.float32)


_fuse_mm = pl.pallas_call(
    _fuse_body,
    grid=(TC_GRID,),
    in_specs=[pl.BlockSpec((NC, TC_BLOCK, D), lambda i: (0, i, 0)),
              pl.BlockSpec((1, 1, TC_BLOCK), lambda i: (i, 0, 0)),
              pl.BlockSpec((T, D), lambda i: (0, 0)),
              pl.BlockSpec((D, D), lambda i: (0, 0))],
    out_specs=pl.BlockSpec((TC_BLOCK, D), lambda i: (i, 0)),
    out_shape=jax.ShapeDtypeStruct((NP, D), jnp.float32),
)


def _final_body(p_ref, nt_ref, tb_ref, o_ref):
    o_ref[...] = p_ref[0] + p_ref[1] + _bias_block(nt_ref, tb_ref)


_final = pl.pallas_call(
    _final_body,
    grid=(TC_GRID,),
    in_specs=[pl.BlockSpec((NC, TC_BLOCK, D), lambda i: (0, i, 0)),
              pl.BlockSpec((1, 1, TC_BLOCK), lambda i: (i, 0, 0)),
              pl.BlockSpec((T, D), lambda i: (0, 0))],
    out_specs=pl.BlockSpec((TC_BLOCK, D), lambda i: (i, 0)),
    out_shape=jax.ShapeDtypeStruct((NP, D), jnp.float32),
)


def kernel(x, edge_index, edge_weight, node_type, W0, tb0, W1, tb1):
    pad = NW * EPT - E
    src = jnp.pad(edge_index[0].astype(jnp.int32), (0, pad)).reshape(NW, NG, G)
    dst = jnp.pad(edge_index[1].astype(jnp.int32), (0, pad)).reshape(NW, NG, G)
    w = jnp.pad(edge_weight.astype(jnp.float32), (0, pad)).reshape(NW, NG, G)
    zi = jnp.zeros((NW, 2, G), jnp.int32)
    src = jnp.concatenate([src, zi], axis=1)
    dst = jnp.concatenate([dst, zi], axis=1)
    w = jnp.concatenate([w, zi.astype(jnp.float32)], axis=1)
    nt = jnp.pad(node_type.astype(jnp.int32), (0, NP - N))
    nt = nt.reshape(TC_GRID, 1, TC_BLOCK)

    h0 = _mm(x, W0)
    p0 = _sc_scatter(h0, src, dst, w)
    h1 = _fuse_mm(p0, nt, tb0, W1)
    p1 = _sc_scatter(h1, src, dst, w)
    return _final(p1, nt, tb1)[:N]
